# baseline (speedup 1.0000x reference)
import functools

import jax
import jax.numpy as jnp
from jax import lax
from jax.experimental import pallas as pl
from jax.experimental.pallas import tpu as pltpu
from jax.experimental.pallas import tpu_sc as plsc

N_NODES = 10000
NPAD = 10240
D = 256
H = 128
E = 160000
NT = 16
CH = 128
NCH = 80
EPAD = NT * NCH * CH
RPT = NPAD // NT

_mesh = plsc.VectorSubcoreMesh(
    core_axis_name="c", subcore_axis_name="s", num_cores=2, num_subcores=NT
)


@functools.partial(
    pl.kernel,
    out_type=jax.ShapeDtypeStruct((2, NPAD), jnp.float32),
    mesh=_mesh,
    scratch_types=[
        pltpu.VMEM((NCH, CH), jnp.int32),
        pltpu.VMEM((CH,), jnp.float32),
        pltpu.VMEM((RPT,), jnp.float32),
        pltpu.VMEM_SHARED((NPAD,), jnp.float32),
    ],
)
def _sc_degrees(idx_hbm, out_hbm, idx_v, ones_v, zero_v, acc):
    c = lax.axis_index("c")
    s = lax.axis_index("s")

    def _setz(k, _):
        zero_v[pl.ds(k * 16, 16)] = jnp.zeros((16,), jnp.float32)
        return _

    lax.fori_loop(0, RPT // 16, _setz, None)

    def _seto(k, _):
        ones_v[pl.ds(k * 16, 16)] = jnp.ones((16,), jnp.float32)
        return _

    lax.fori_loop(0, CH // 16, _seto, None)

    pltpu.sync_copy(idx_hbm.at[c, s], idx_v)
    pltpu.sync_copy(zero_v, acc.at[pl.ds(s * RPT, RPT)])
    plsc.subcore_barrier()

    def _body(j, _):
        pltpu.sync_copy(ones_v, acc.at[idx_v.at[j]], add=True)
        return _

    lax.fori_loop(0, NCH, _body, None)
    plsc.subcore_barrier()
    pltpu.sync_copy(acc.at[pl.ds(s * RPT, RPT)], out_hbm.at[c, pl.ds(s * RPT, RPT)])


def _make_sc_aggregate(dtype):
  return functools.partial(
    pl.kernel,
    out_type=jax.ShapeDtypeStruct((2, NPAD, H), dtype),
    mesh=_mesh,
    scratch_types=[
        pltpu.VMEM((2, 4, 2, CH), jnp.int32),
        pltpu.VMEM((CH, H), dtype),
        pltpu.VMEM((CH, H), dtype),
        pltpu.SemaphoreType.DMA,
        pltpu.SemaphoreType.DMA,
        pltpu.SemaphoreType.DMA,
        pltpu.SemaphoreType.DMA,
        pltpu.SemaphoreType.DMA,
        pltpu.SemaphoreType.DMA,
        pltpu.VMEM_SHARED((NPAD, H), dtype),
    ],
  )(_sc_aggregate_body)


def _sc_aggregate_body(table_hbm, init_hbm, sridx_hbm, out_hbm,
                       idxr, rows0_v, rows1_v,
                       isem0, isem1, gsem0, gsem1,
                       ssem0, ssem1, acc):
    c = lax.axis_index("c")
    s = lax.axis_index("s")
    pltpu.sync_copy(init_hbm.at[c].at[pl.ds(s * RPT, RPT)],
                    acc.at[pl.ds(s * RPT, RPT)])

    table = table_hbm.at[c]
    isems = (isem0, isem1)
    rows = (rows0_v, rows1_v)
    gsems = (gsem0, gsem1)
    ssems = (ssem0, ssem1)

    def _idx_fetch(sl, j):
        pltpu.async_copy(sridx_hbm.at[s, pl.ds(j, 4)], idxr.at[sl],
                         isems[sl])

    def _idx_wait(sl):
        pltpu.make_async_copy(sridx_hbm.at[s, pl.ds(0, 4)], idxr.at[sl],
                              isems[sl]).wait()

    def _gather(sl, pos, k):
        pltpu.async_copy(table.at[idxr.at[sl, pos, 0]], rows[k], gsems[k])

    def _gather_wait(k):
        pltpu.make_async_copy(table.at[idxr.at[0, 0, 0]], rows[k],
                              gsems[k]).wait()

    def _scatter(sl, pos, k):
        pltpu.async_copy(rows[k], acc.at[idxr.at[sl, pos, 1]], ssems[k],
                         add=True)

    def _scatter_wait(k):
        pltpu.make_async_copy(rows[k], acc.at[idxr.at[0, 0, 1]],
                              ssems[k]).wait()

    _idx_fetch(0, 0)
    _idx_fetch(1, 4)
    plsc.subcore_barrier()
    _idx_wait(0)
    _gather(0, 0, 0)

    def _body(i, _):
        j = 8 * i
        for q in range(8):
            jq = j + q
            sl, pos, k = q // 4, q % 4, q % 2
            nk = (q + 1) % 2
            if q == 0:
                @pl.when(j > 0)
                def _():
                    _scatter_wait(nk)
                    _idx_fetch(1, j + 4)
            else:
                _scatter_wait(nk)
                if q == 4:
                    @pl.when(j + 8 < NCH)
                    def _():
                        _idx_fetch(0, j + 8)
            if q == 3:
                _idx_wait(1)
                _gather(1, 0, nk)
            elif q < 7:
                _gather(sl, pos + 1, nk)
            else:
                @pl.when(j + 8 < NCH)
                def _():
                    _idx_wait(0)
                    _gather(0, 0, nk)
            _gather_wait(k)
            _scatter(sl, pos, k)
        return _

    lax.fori_loop(0, NCH // 8, _body, None)
    _scatter_wait(1)
    plsc.subcore_barrier()
    pltpu.sync_copy(acc.at[pl.ds(s * RPT, RPT)],
                    out_hbm.at[c].at[pl.ds(s * RPT, RPT)])


_sc_aggregate_f32 = _make_sc_aggregate(jnp.float32)


_BR = 400
_GRID = N_NODES // _BR


def _tc1_body(x_ref, w_ref, b_ref, degs_ref, out_ref):
    h = jnp.dot(x_ref[...], w_ref[...], preferred_element_type=jnp.float32)
    h = h + b_ref[...]
    h = h * lax.rsqrt(jnp.maximum(degs_ref[...] + 1.0, 1.0))
    out_ref[0] = h[:, :H]
    out_ref[1] = h[:, H:]


def _tc2_body(a_ref, w_ref, b_ref, degr_ref, degs_ref, out_ref):
    t = jnp.concatenate([a_ref[0], a_ref[1]], axis=1)
    t = t * lax.rsqrt(jnp.maximum(degr_ref[...] + 1.0, 1.0))
    h = jnp.dot(t, w_ref[...], preferred_element_type=jnp.float32)
    h = h + b_ref[...]
    h = h * lax.rsqrt(jnp.maximum(degs_ref[...], 1.0))
    out_ref[0] = h[:, :H]
    out_ref[1] = h[:, H:]


def _tc3_body(a_ref, degr_ref, out_ref):
    t = jnp.concatenate([a_ref[0], a_ref[1]], axis=1)
    out_ref[...] = t * lax.rsqrt(jnp.maximum(degr_ref[...], 1.0))


_halves_spec = pl.BlockSpec((2, _BR, H), lambda i: (0, i, 0))
_rows_spec = pl.BlockSpec((_BR, D), lambda i: (i, 0))
_deg_spec = pl.BlockSpec((_BR, 1), lambda i: (i, 0))
_w_spec = pl.BlockSpec((D, D), lambda i: (0, 0))
_b_spec = pl.BlockSpec((1, D), lambda i: (0, 0))

_tc1 = pl.pallas_call(
    _tc1_body,
    grid=(_GRID,),
    in_specs=[_rows_spec, _w_spec, _b_spec, _deg_spec],
    out_specs=_halves_spec,
    out_shape=jax.ShapeDtypeStruct((2, NPAD, H), jnp.float32),
)

_tc2 = pl.pallas_call(
    _tc2_body,
    grid=(_GRID,),
    in_specs=[_halves_spec, _w_spec, _b_spec, _deg_spec, _deg_spec],
    out_specs=_halves_spec,
    out_shape=jax.ShapeDtypeStruct((2, NPAD, H), jnp.float32),
)

_tc3 = pl.pallas_call(
    _tc3_body,
    grid=(_GRID,),
    in_specs=[_halves_spec, _deg_spec],
    out_specs=_rows_spec,
    out_shape=jax.ShapeDtypeStruct((N_NODES, D), jnp.float32),
)


def kernel(x, edge_index, W1, b1, W2, b2):
    ei = edge_index.astype(jnp.int32)
    senders, receivers = ei[0], ei[1]
    npad_e = EPAD - E
    dummy = jnp.full((npad_e,), N_NODES, dtype=jnp.int32)
    zpad = jnp.zeros((npad_e,), dtype=jnp.int32)

    hist_idx = jnp.stack([
        jnp.concatenate([senders, dummy]),
        jnp.concatenate([receivers, dummy]),
    ]).reshape(2, NT, NCH, CH)
    sidx = jnp.concatenate([senders, zpad]).reshape(NT, NCH, CH)
    ridx = jnp.concatenate([receivers, dummy]).reshape(NT, NCH, CH)
    sridx = jnp.stack([sidx, ridx], axis=2)

    b1r = b1.reshape(1, D)
    b2r = b2.reshape(1, D)

    deg = _sc_degrees(hist_idx)
    deg_s = deg[0].reshape(NPAD, 1)
    deg_r = deg[1].reshape(NPAD, 1)

    hs = _tc1(x, W1, b1r, deg_s)
    agg1 = _sc_aggregate_f32(hs, hs, sridx)
    h2s = _tc2(agg1, W2, b2r, deg_r, deg_s)
    zeros_init = jnp.zeros((2, NPAD, H), dtype=jnp.float32)
    agg2 = _sc_aggregate_f32(h2s, zeros_init, sridx)
    return _tc3(agg2, deg_r)


# speedup vs baseline: 1.0718x; 1.0718x over previous
import functools

import jax
import jax.numpy as jnp
from jax import lax
from jax.experimental import pallas as pl
from jax.experimental.pallas import tpu as pltpu
from jax.experimental.pallas import tpu_sc as plsc

N_NODES = 10000
NPAD = 10240
D = 256
H = 128
E = 160000
NT = 16
CH = 128
NCH = 80
EPAD = NT * NCH * CH
RPT = NPAD // NT

_mesh = plsc.VectorSubcoreMesh(
    core_axis_name="c", subcore_axis_name="s", num_cores=2, num_subcores=NT
)


@functools.partial(
    pl.kernel,
    out_type=jax.ShapeDtypeStruct((2, NPAD), jnp.float32),
    mesh=_mesh,
    scratch_types=[
        pltpu.VMEM((NCH, CH), jnp.int32),
        pltpu.VMEM((CH,), jnp.float32),
        pltpu.VMEM((RPT,), jnp.float32),
        pltpu.VMEM_SHARED((NPAD,), jnp.float32),
    ],
)
def _sc_degrees(idx_hbm, out_hbm, idx_v, ones_v, zero_v, acc):
    c = lax.axis_index("c")
    s = lax.axis_index("s")

    def _setz(k, _):
        zero_v[pl.ds(k * 16, 16)] = jnp.zeros((16,), jnp.float32)
        return _

    lax.fori_loop(0, RPT // 16, _setz, None)

    def _seto(k, _):
        ones_v[pl.ds(k * 16, 16)] = jnp.ones((16,), jnp.float32)
        return _

    lax.fori_loop(0, CH // 16, _seto, None)

    pltpu.sync_copy(idx_hbm.at[c, s], idx_v)
    pltpu.sync_copy(zero_v, acc.at[pl.ds(s * RPT, RPT)])
    plsc.subcore_barrier()

    def _body(j, _):
        pltpu.sync_copy(ones_v, acc.at[idx_v.at[j]], add=True)
        return _

    lax.fori_loop(0, NCH, _body, None)
    plsc.subcore_barrier()
    pltpu.sync_copy(acc.at[pl.ds(s * RPT, RPT)], out_hbm.at[c, pl.ds(s * RPT, RPT)])


def _make_sc_aggregate(dtype):
  return functools.partial(
    pl.kernel,
    out_type=jax.ShapeDtypeStruct((2, NPAD, H), dtype),
    mesh=_mesh,
    scratch_types=[
        pltpu.VMEM((2, 4, 2, CH), jnp.int32),
        pltpu.VMEM((CH, H), dtype),
        pltpu.VMEM((CH, H), dtype),
        pltpu.SemaphoreType.DMA,
        pltpu.SemaphoreType.DMA,
        pltpu.SemaphoreType.DMA,
        pltpu.SemaphoreType.DMA,
        pltpu.SemaphoreType.DMA,
        pltpu.SemaphoreType.DMA,
        pltpu.VMEM_SHARED((NPAD, H), dtype),
    ],
  )(_sc_aggregate_body)


def _sc_aggregate_body(table_hbm, init_hbm, sridx_hbm, out_hbm,
                       idxr, rows0_v, rows1_v,
                       isem0, isem1, gsem0, gsem1,
                       ssem0, ssem1, acc):
    c = lax.axis_index("c")
    s = lax.axis_index("s")
    pltpu.sync_copy(init_hbm.at[c].at[pl.ds(s * RPT, RPT)],
                    acc.at[pl.ds(s * RPT, RPT)])

    table = table_hbm.at[c]
    isems = (isem0, isem1)
    rows = (rows0_v, rows1_v)
    gsems = (gsem0, gsem1)
    ssems = (ssem0, ssem1)

    def _idx_fetch(sl, j):
        pltpu.async_copy(sridx_hbm.at[s, pl.ds(j, 4)], idxr.at[sl],
                         isems[sl])

    def _idx_wait(sl):
        pltpu.make_async_copy(sridx_hbm.at[s, pl.ds(0, 4)], idxr.at[sl],
                              isems[sl]).wait()

    def _gather(sl, pos, k):
        pltpu.async_copy(table.at[idxr.at[sl, pos, 0]], rows[k], gsems[k])

    def _gather_wait(k):
        pltpu.make_async_copy(table.at[idxr.at[0, 0, 0]], rows[k],
                              gsems[k]).wait()

    def _scatter(sl, pos, k):
        pltpu.async_copy(rows[k], acc.at[idxr.at[sl, pos, 1]], ssems[k],
                         add=True)

    def _scatter_wait(k):
        pltpu.make_async_copy(rows[k], acc.at[idxr.at[0, 0, 1]],
                              ssems[k]).wait()

    _idx_fetch(0, 0)
    _idx_fetch(1, 4)
    plsc.subcore_barrier()
    _idx_wait(0)
    _gather(0, 0, 0)

    def _body(i, _):
        j = 8 * i
        for q in range(8):
            jq = j + q
            sl, pos, k = q // 4, q % 4, q % 2
            nk = (q + 1) % 2
            if q == 0:
                @pl.when(j > 0)
                def _():
                    _scatter_wait(nk)
                    _idx_fetch(1, j + 4)
            else:
                _scatter_wait(nk)
                if q == 4:
                    @pl.when(j + 8 < NCH)
                    def _():
                        _idx_fetch(0, j + 8)
            if q == 3:
                _idx_wait(1)
                _gather(1, 0, nk)
            elif q < 7:
                _gather(sl, pos + 1, nk)
            else:
                @pl.when(j + 8 < NCH)
                def _():
                    _idx_wait(0)
                    _gather(0, 0, nk)
            _gather_wait(k)
            _scatter(sl, pos, k)
        return _

    lax.fori_loop(0, NCH // 8, _body, None)
    _scatter_wait(1)
    plsc.subcore_barrier()
    pltpu.sync_copy(acc.at[pl.ds(s * RPT, RPT)],
                    out_hbm.at[c].at[pl.ds(s * RPT, RPT)])


_sc_aggregate_f32 = _make_sc_aggregate(jnp.float32)


_BR = 1024
_GRID = NPAD // _BR


def _tc1_body(x_ref, w_ref, b_ref, degs_ref, out_ref):
    h = jnp.dot(x_ref[...], w_ref[...], preferred_element_type=jnp.float32)
    h = h + b_ref[...]
    h = h * lax.rsqrt(jnp.maximum(degs_ref[...] + 1.0, 1.0))
    out_ref[0] = h[:, :H]
    out_ref[1] = h[:, H:]


def _tc2_body(a_ref, w_ref, b_ref, degr_ref, degs_ref, out_ref):
    t = jnp.concatenate([a_ref[0], a_ref[1]], axis=1)
    t = t * lax.rsqrt(jnp.maximum(degr_ref[...] + 1.0, 1.0))
    h = jnp.dot(t, w_ref[...], preferred_element_type=jnp.float32)
    h = h + b_ref[...]
    h = h * lax.rsqrt(jnp.maximum(degs_ref[...], 1.0))
    out_ref[0] = h[:, :H]
    out_ref[1] = h[:, H:]


def _tc3_body(a_ref, degr_ref, out_ref):
    t = jnp.concatenate([a_ref[0], a_ref[1]], axis=1)
    out_ref[...] = t * lax.rsqrt(jnp.maximum(degr_ref[...], 1.0))


_halves_spec = pl.BlockSpec((2, _BR, H), lambda i: (0, i, 0))
_rows_spec = pl.BlockSpec((_BR, D), lambda i: (i, 0))
_deg_spec = pl.BlockSpec((_BR, 1), lambda i: (i, 0))
_w_spec = pl.BlockSpec((D, D), lambda i: (0, 0))
_b_spec = pl.BlockSpec((1, D), lambda i: (0, 0))

_tc1 = pl.pallas_call(
    _tc1_body,
    grid=(_GRID,),
    in_specs=[_rows_spec, _w_spec, _b_spec, _deg_spec],
    out_specs=_halves_spec,
    out_shape=jax.ShapeDtypeStruct((2, NPAD, H), jnp.float32),
)

_tc2 = pl.pallas_call(
    _tc2_body,
    grid=(_GRID,),
    in_specs=[_halves_spec, _w_spec, _b_spec, _deg_spec, _deg_spec],
    out_specs=_halves_spec,
    out_shape=jax.ShapeDtypeStruct((2, NPAD, H), jnp.float32),
)

_tc3 = pl.pallas_call(
    _tc3_body,
    grid=(_GRID,),
    in_specs=[_halves_spec, _deg_spec],
    out_specs=_rows_spec,
    out_shape=jax.ShapeDtypeStruct((NPAD, D), jnp.float32),
)


def kernel(x, edge_index, W1, b1, W2, b2):
    ei = edge_index.astype(jnp.int32)
    senders, receivers = ei[0], ei[1]
    npad_e = EPAD - E
    dummy = jnp.full((npad_e,), N_NODES, dtype=jnp.int32)
    zpad = jnp.zeros((npad_e,), dtype=jnp.int32)

    hist_idx = jnp.stack([
        jnp.concatenate([senders, dummy]),
        jnp.concatenate([receivers, dummy]),
    ]).reshape(2, NT, NCH, CH)
    sidx = jnp.concatenate([senders, zpad]).reshape(NT, NCH, CH)
    ridx = jnp.concatenate([receivers, dummy]).reshape(NT, NCH, CH)
    sridx = jnp.stack([sidx, ridx], axis=2)

    x_pad = jnp.concatenate(
        [x, jnp.zeros((NPAD - N_NODES, D), dtype=jnp.float32)], axis=0
    )
    b1r = b1.reshape(1, D)
    b2r = b2.reshape(1, D)

    deg = _sc_degrees(hist_idx)
    deg_s = deg[0].reshape(NPAD, 1)
    deg_r = deg[1].reshape(NPAD, 1)

    hs = _tc1(x_pad, W1, b1r, deg_s)
    agg1 = _sc_aggregate_f32(hs, hs, sridx)
    h2s = _tc2(agg1, W2, b2r, deg_r, deg_s)
    zeros_init = jnp.zeros((2, NPAD, H), dtype=jnp.float32)
    agg2 = _sc_aggregate_f32(h2s, zeros_init, sridx)
    out = _tc3(agg2, deg_r)
    return out[:N_NODES]


# speedup vs baseline: 1.0784x; 1.0062x over previous
import functools

import jax
import jax.numpy as jnp
from jax import lax
from jax.experimental import pallas as pl
from jax.experimental.pallas import tpu as pltpu
from jax.experimental.pallas import tpu_sc as plsc

N_NODES = 10000
NPAD = 10240
D = 256
H = 128
E = 160000
NT = 16
CH = 128
NCH = 80
EPAD = NT * NCH * CH
RPT = NPAD // NT

_mesh = plsc.VectorSubcoreMesh(
    core_axis_name="c", subcore_axis_name="s", num_cores=2, num_subcores=NT
)


@functools.partial(
    pl.kernel,
    out_type=jax.ShapeDtypeStruct((2, NPAD), jnp.float32),
    mesh=_mesh,
    scratch_types=[
        pltpu.VMEM((NCH, CH), jnp.int32),
        pltpu.VMEM((CH,), jnp.float32),
        pltpu.VMEM((RPT,), jnp.float32),
        pltpu.VMEM_SHARED((NPAD,), jnp.float32),
    ],
)
def _sc_degrees(idx_hbm, out_hbm, idx_v, ones_v, zero_v, acc):
    c = lax.axis_index("c")
    s = lax.axis_index("s")

    def _setz(k, _):
        zero_v[pl.ds(k * 16, 16)] = jnp.zeros((16,), jnp.float32)
        return _

    lax.fori_loop(0, RPT // 16, _setz, None)

    def _seto(k, _):
        ones_v[pl.ds(k * 16, 16)] = jnp.ones((16,), jnp.float32)
        return _

    lax.fori_loop(0, CH // 16, _seto, None)

    pltpu.sync_copy(idx_hbm.at[c, s], idx_v)
    pltpu.sync_copy(zero_v, acc.at[pl.ds(s * RPT, RPT)])
    plsc.subcore_barrier()

    def _body(j, _):
        pltpu.sync_copy(ones_v, acc.at[idx_v.at[j]], add=True)
        return _

    lax.fori_loop(0, NCH, _body, None)
    plsc.subcore_barrier()
    pltpu.sync_copy(acc.at[pl.ds(s * RPT, RPT)], out_hbm.at[c, pl.ds(s * RPT, RPT)])


def _make_sc_aggregate(dtype):
  return functools.partial(
    pl.kernel,
    out_type=jax.ShapeDtypeStruct((2, NPAD, H), dtype),
    mesh=_mesh,
    scratch_types=[
        pltpu.VMEM((2, 4, 2, CH), jnp.int32),
        pltpu.VMEM((CH, H), dtype),
        pltpu.VMEM((CH, H), dtype),
        pltpu.SemaphoreType.DMA,
        pltpu.SemaphoreType.DMA,
        pltpu.SemaphoreType.DMA,
        pltpu.SemaphoreType.DMA,
        pltpu.SemaphoreType.DMA,
        pltpu.SemaphoreType.DMA,
        pltpu.VMEM_SHARED((NPAD, H), dtype),
    ],
  )(_sc_aggregate_body)


def _sc_aggregate_body(table_hbm, init_hbm, sridx_hbm, out_hbm,
                       idxr, rows0_v, rows1_v,
                       isem0, isem1, gsem0, gsem1,
                       ssem0, ssem1, acc):
    c = lax.axis_index("c")
    s = lax.axis_index("s")
    pltpu.sync_copy(init_hbm.at[c].at[pl.ds(s * RPT, RPT)],
                    acc.at[pl.ds(s * RPT, RPT)])

    table = table_hbm.at[c]
    isems = (isem0, isem1)
    rows = (rows0_v, rows1_v)
    gsems = (gsem0, gsem1)
    ssems = (ssem0, ssem1)

    def _idx_fetch(sl, j):
        pltpu.async_copy(sridx_hbm.at[s, pl.ds(j, 4)], idxr.at[sl],
                         isems[sl])

    def _idx_wait(sl):
        pltpu.make_async_copy(sridx_hbm.at[s, pl.ds(0, 4)], idxr.at[sl],
                              isems[sl]).wait()

    def _gather(sl, pos, k):
        pltpu.async_copy(table.at[idxr.at[sl, pos, 0]], rows[k], gsems[k])

    def _gather_wait(k):
        pltpu.make_async_copy(table.at[idxr.at[0, 0, 0]], rows[k],
                              gsems[k]).wait()

    def _scatter(sl, pos, k):
        pltpu.async_copy(rows[k], acc.at[idxr.at[sl, pos, 1]], ssems[k],
                         add=True)

    def _scatter_wait(k):
        pltpu.make_async_copy(rows[k], acc.at[idxr.at[0, 0, 1]],
                              ssems[k]).wait()

    _idx_fetch(0, 0)
    _idx_fetch(1, 4)
    plsc.subcore_barrier()
    _idx_wait(0)
    _gather(0, 0, 0)

    def _body(i, _):
        j = 8 * i
        for q in range(8):
            jq = j + q
            sl, pos, k = q // 4, q % 4, q % 2
            nk = (q + 1) % 2
            if q == 0:
                @pl.when(j > 0)
                def _():
                    _scatter_wait(nk)
                    _idx_fetch(1, j + 4)
            else:
                _scatter_wait(nk)
                if q == 4:
                    @pl.when(j + 8 < NCH)
                    def _():
                        _idx_fetch(0, j + 8)
            if q == 3:
                _idx_wait(1)
                _gather(1, 0, nk)
            elif q < 7:
                _gather(sl, pos + 1, nk)
            else:
                @pl.when(j + 8 < NCH)
                def _():
                    _idx_wait(0)
                    _gather(0, 0, nk)
            _gather_wait(k)
            _scatter(sl, pos, k)
        return _

    lax.fori_loop(0, NCH // 8, _body, None)
    _scatter_wait(1)
    plsc.subcore_barrier()
    pltpu.sync_copy(acc.at[pl.ds(s * RPT, RPT)],
                    out_hbm.at[c].at[pl.ds(s * RPT, RPT)])


_sc_aggregate_f32 = _make_sc_aggregate(jnp.float32)


_BR = 2048
_GRID = NPAD // _BR


def _tc1_body(x_ref, w_ref, b_ref, degs_ref, out_ref):
    h = jnp.dot(x_ref[...], w_ref[...], preferred_element_type=jnp.float32)
    h = h + b_ref[...]
    h = h * lax.rsqrt(jnp.maximum(degs_ref[...] + 1.0, 1.0))
    out_ref[0] = h[:, :H]
    out_ref[1] = h[:, H:]


def _tc2_body(a_ref, w_ref, b_ref, degr_ref, degs_ref, out_ref):
    t = jnp.concatenate([a_ref[0], a_ref[1]], axis=1)
    t = t * lax.rsqrt(jnp.maximum(degr_ref[...] + 1.0, 1.0))
    h = jnp.dot(t, w_ref[...], preferred_element_type=jnp.float32)
    h = h + b_ref[...]
    h = h * lax.rsqrt(jnp.maximum(degs_ref[...], 1.0))
    out_ref[0] = h[:, :H]
    out_ref[1] = h[:, H:]


def _tc3_body(a_ref, degr_ref, out_ref):
    t = jnp.concatenate([a_ref[0], a_ref[1]], axis=1)
    out_ref[...] = t * lax.rsqrt(jnp.maximum(degr_ref[...], 1.0))


_halves_spec = pl.BlockSpec((2, _BR, H), lambda i: (0, i, 0))
_rows_spec = pl.BlockSpec((_BR, D), lambda i: (i, 0))
_deg_spec = pl.BlockSpec((_BR, 1), lambda i: (i, 0))
_w_spec = pl.BlockSpec((D, D), lambda i: (0, 0))
_b_spec = pl.BlockSpec((1, D), lambda i: (0, 0))

_tc1 = pl.pallas_call(
    _tc1_body,
    grid=(_GRID,),
    in_specs=[_rows_spec, _w_spec, _b_spec, _deg_spec],
    out_specs=_halves_spec,
    out_shape=jax.ShapeDtypeStruct((2, NPAD, H), jnp.float32),
)

_tc2 = pl.pallas_call(
    _tc2_body,
    grid=(_GRID,),
    in_specs=[_halves_spec, _w_spec, _b_spec, _deg_spec, _deg_spec],
    out_specs=_halves_spec,
    out_shape=jax.ShapeDtypeStruct((2, NPAD, H), jnp.float32),
)

_tc3 = pl.pallas_call(
    _tc3_body,
    grid=(_GRID,),
    in_specs=[_halves_spec, _deg_spec],
    out_specs=_rows_spec,
    out_shape=jax.ShapeDtypeStruct((NPAD, D), jnp.float32),
)


def kernel(x, edge_index, W1, b1, W2, b2):
    ei = edge_index.astype(jnp.int32)
    senders, receivers = ei[0], ei[1]
    npad_e = EPAD - E
    dummy = jnp.full((npad_e,), N_NODES, dtype=jnp.int32)
    zpad = jnp.zeros((npad_e,), dtype=jnp.int32)

    hist_idx = jnp.stack([
        jnp.concatenate([senders, dummy]),
        jnp.concatenate([receivers, dummy]),
    ]).reshape(2, NT, NCH, CH)
    sidx = jnp.concatenate([senders, zpad]).reshape(NT, NCH, CH)
    ridx = jnp.concatenate([receivers, dummy]).reshape(NT, NCH, CH)
    sridx = jnp.stack([sidx, ridx], axis=2)

    x_pad = jnp.concatenate(
        [x, jnp.zeros((NPAD - N_NODES, D), dtype=jnp.float32)], axis=0
    )
    b1r = b1.reshape(1, D)
    b2r = b2.reshape(1, D)

    deg = _sc_degrees(hist_idx)
    deg_s = deg[0].reshape(NPAD, 1)
    deg_r = deg[1].reshape(NPAD, 1)

    hs = _tc1(x_pad, W1, b1r, deg_s)
    agg1 = _sc_aggregate_f32(hs, hs, sridx)
    h2s = _tc2(agg1, W2, b2r, deg_r, deg_s)
    zeros_init = jnp.zeros((2, NPAD, H), dtype=jnp.float32)
    agg2 = _sc_aggregate_f32(h2s, zeros_init, sridx)
    out = _tc3(agg2, deg_r)
    return out[:N_NODES]


# speedup vs baseline: 1.0785x; 1.0001x over previous
import functools

import jax
import jax.numpy as jnp
from jax import lax
from jax.experimental import pallas as pl
from jax.experimental.pallas import tpu as pltpu
from jax.experimental.pallas import tpu_sc as plsc

N_NODES = 10000
NPAD = 10240
D = 256
H = 128
E = 160000
NT = 16
CH = 128
NCH = 80
EPAD = NT * NCH * CH
RPT = NPAD // NT

_mesh = plsc.VectorSubcoreMesh(
    core_axis_name="c", subcore_axis_name="s", num_cores=2, num_subcores=NT
)


@functools.partial(
    pl.kernel,
    out_type=jax.ShapeDtypeStruct((2, NPAD), jnp.float32),
    mesh=_mesh,
    scratch_types=[
        pltpu.VMEM((NCH, CH), jnp.int32),
        pltpu.VMEM((CH,), jnp.float32),
        pltpu.VMEM((RPT,), jnp.float32),
        pltpu.VMEM_SHARED((NPAD,), jnp.float32),
    ],
)
def _sc_degrees(idx_hbm, out_hbm, idx_v, ones_v, zero_v, acc):
    c = lax.axis_index("c")
    s = lax.axis_index("s")

    def _setz(k, _):
        zero_v[pl.ds(k * 16, 16)] = jnp.zeros((16,), jnp.float32)
        return _

    lax.fori_loop(0, RPT // 16, _setz, None)

    def _seto(k, _):
        ones_v[pl.ds(k * 16, 16)] = jnp.ones((16,), jnp.float32)
        return _

    lax.fori_loop(0, CH // 16, _seto, None)

    pltpu.sync_copy(idx_hbm.at[c, s], idx_v)
    pltpu.sync_copy(zero_v, acc.at[pl.ds(s * RPT, RPT)])
    plsc.subcore_barrier()

    def _body(j, _):
        pltpu.sync_copy(ones_v, acc.at[idx_v.at[j]], add=True)
        return _

    lax.fori_loop(0, NCH, _body, None)
    plsc.subcore_barrier()
    pltpu.sync_copy(acc.at[pl.ds(s * RPT, RPT)], out_hbm.at[c, pl.ds(s * RPT, RPT)])


def _make_sc_aggregate(dtype):
  return functools.partial(
    pl.kernel,
    out_type=jax.ShapeDtypeStruct((2, NPAD, H), dtype),
    mesh=_mesh,
    scratch_types=[
        pltpu.VMEM((2, 4, 2, CH), jnp.int32),
        pltpu.VMEM((CH, H), dtype),
        pltpu.VMEM((CH, H), dtype),
        pltpu.SemaphoreType.DMA,
        pltpu.SemaphoreType.DMA,
        pltpu.SemaphoreType.DMA,
        pltpu.SemaphoreType.DMA,
        pltpu.SemaphoreType.DMA,
        pltpu.SemaphoreType.DMA,
        pltpu.VMEM_SHARED((NPAD, H), dtype),
    ],
  )(_sc_aggregate_body)


def _sc_aggregate_body(table_hbm, init_hbm, sridx_hbm, out_hbm,
                       idxr, rows0_v, rows1_v,
                       isem0, isem1, gsem0, gsem1,
                       ssem0, ssem1, acc):
    c = lax.axis_index("c")
    s = lax.axis_index("s")
    pltpu.sync_copy(init_hbm.at[c].at[pl.ds(s * RPT, RPT)],
                    acc.at[pl.ds(s * RPT, RPT)])

    table = table_hbm.at[c]
    isems = (isem0, isem1)
    rows = (rows0_v, rows1_v)
    gsems = (gsem0, gsem1)
    ssems = (ssem0, ssem1)

    def _idx_fetch(sl, j):
        pltpu.async_copy(sridx_hbm.at[s, pl.ds(j, 4)], idxr.at[sl],
                         isems[sl])

    def _idx_wait(sl):
        pltpu.make_async_copy(sridx_hbm.at[s, pl.ds(0, 4)], idxr.at[sl],
                              isems[sl]).wait()

    def _gather(sl, pos, k):
        pltpu.async_copy(table.at[idxr.at[sl, pos, 0]], rows[k], gsems[k])

    def _gather_wait(k):
        pltpu.make_async_copy(table.at[idxr.at[0, 0, 0]], rows[k],
                              gsems[k]).wait()

    def _scatter(sl, pos, k):
        pltpu.async_copy(rows[k], acc.at[idxr.at[sl, pos, 1]], ssems[k],
                         add=True)

    def _scatter_wait(k):
        pltpu.make_async_copy(rows[k], acc.at[idxr.at[0, 0, 1]],
                              ssems[k]).wait()

    _idx_fetch(0, 0)
    _idx_fetch(1, 4)
    plsc.subcore_barrier()
    _idx_wait(0)
    _gather(0, 0, 0)

    def _body(i, _):
        j = 8 * i
        for q in range(8):
            jq = j + q
            sl, pos, k = q // 4, q % 4, q % 2
            nk = (q + 1) % 2
            if q == 0:
                @pl.when(j > 0)
                def _():
                    _scatter_wait(nk)
                    _idx_fetch(1, j + 4)
            else:
                _scatter_wait(nk)
                if q == 4:
                    @pl.when(j + 8 < NCH)
                    def _():
                        _idx_fetch(0, j + 8)
            if q == 3:
                _idx_wait(1)
                _gather(1, 0, nk)
            elif q < 7:
                _gather(sl, pos + 1, nk)
            else:
                @pl.when(j + 8 < NCH)
                def _():
                    _idx_wait(0)
                    _gather(0, 0, nk)
            _gather_wait(k)
            _scatter(sl, pos, k)
        return _

    lax.fori_loop(0, NCH // 8, _body, None)
    _scatter_wait(1)
    plsc.subcore_barrier()
    pltpu.sync_copy(acc.at[pl.ds(s * RPT, RPT)],
                    out_hbm.at[c].at[pl.ds(s * RPT, RPT)])


_sc_aggregate_f32 = _make_sc_aggregate(jnp.float32)


_BR = 2048
_GRID = NPAD // _BR


def _tc1_body(x_ref, w_ref, b_ref, degs_ref, out_ref):
    h = jnp.dot(x_ref[...].astype(jnp.bfloat16), w_ref[...].astype(jnp.bfloat16),
                preferred_element_type=jnp.float32)
    h = h + b_ref[...]
    h = h * lax.rsqrt(jnp.maximum(degs_ref[...] + 1.0, 1.0))
    out_ref[0] = h[:, :H]
    out_ref[1] = h[:, H:]


def _tc2_body(a_ref, w_ref, b_ref, degr_ref, degs_ref, out_ref):
    t = jnp.concatenate([a_ref[0], a_ref[1]], axis=1)
    t = t * lax.rsqrt(jnp.maximum(degr_ref[...] + 1.0, 1.0))
    h = jnp.dot(t.astype(jnp.bfloat16), w_ref[...].astype(jnp.bfloat16),
                preferred_element_type=jnp.float32)
    h = h + b_ref[...]
    h = h * lax.rsqrt(jnp.maximum(degs_ref[...], 1.0))
    out_ref[0] = h[:, :H]
    out_ref[1] = h[:, H:]


def _tc3_body(a_ref, degr_ref, out_ref):
    t = jnp.concatenate([a_ref[0], a_ref[1]], axis=1)
    out_ref[...] = t * lax.rsqrt(jnp.maximum(degr_ref[...], 1.0))


_halves_spec = pl.BlockSpec((2, _BR, H), lambda i: (0, i, 0))
_rows_spec = pl.BlockSpec((_BR, D), lambda i: (i, 0))
_deg_spec = pl.BlockSpec((_BR, 1), lambda i: (i, 0))
_w_spec = pl.BlockSpec((D, D), lambda i: (0, 0))
_b_spec = pl.BlockSpec((1, D), lambda i: (0, 0))

_tc1 = pl.pallas_call(
    _tc1_body,
    grid=(_GRID,),
    in_specs=[_rows_spec, _w_spec, _b_spec, _deg_spec],
    out_specs=_halves_spec,
    out_shape=jax.ShapeDtypeStruct((2, NPAD, H), jnp.float32),
)

_tc2 = pl.pallas_call(
    _tc2_body,
    grid=(_GRID,),
    in_specs=[_halves_spec, _w_spec, _b_spec, _deg_spec, _deg_spec],
    out_specs=_halves_spec,
    out_shape=jax.ShapeDtypeStruct((2, NPAD, H), jnp.float32),
)

_tc3 = pl.pallas_call(
    _tc3_body,
    grid=(_GRID,),
    in_specs=[_halves_spec, _deg_spec],
    out_specs=_rows_spec,
    out_shape=jax.ShapeDtypeStruct((NPAD, D), jnp.float32),
)


def kernel(x, edge_index, W1, b1, W2, b2):
    ei = edge_index.astype(jnp.int32)
    senders, receivers = ei[0], ei[1]
    npad_e = EPAD - E
    dummy = jnp.full((npad_e,), N_NODES, dtype=jnp.int32)
    zpad = jnp.zeros((npad_e,), dtype=jnp.int32)

    hist_idx = jnp.stack([
        jnp.concatenate([senders, dummy]),
        jnp.concatenate([receivers, dummy]),
    ]).reshape(2, NT, NCH, CH)
    sidx = jnp.concatenate([senders, zpad]).reshape(NT, NCH, CH)
    ridx = jnp.concatenate([receivers, dummy]).reshape(NT, NCH, CH)
    sridx = jnp.stack([sidx, ridx], axis=2)

    x_pad = jnp.concatenate(
        [x, jnp.zeros((NPAD - N_NODES, D), dtype=jnp.float32)], axis=0
    )
    b1r = b1.reshape(1, D)
    b2r = b2.reshape(1, D)

    deg = _sc_degrees(hist_idx)
    deg_s = deg[0].reshape(NPAD, 1)
    deg_r = deg[1].reshape(NPAD, 1)

    hs = _tc1(x_pad, W1, b1r, deg_s)
    agg1 = _sc_aggregate_f32(hs, hs, sridx)
    h2s = _tc2(agg1, W2, b2r, deg_r, deg_s)
    zeros_init = jnp.zeros((2, NPAD, H), dtype=jnp.float32)
    agg2 = _sc_aggregate_f32(h2s, zeros_init, sridx)
    out = _tc3(agg2, deg_r)
    return out[:N_NODES]


# speedup vs baseline: 1.0797x; 1.0011x over previous
import functools

import jax
import jax.numpy as jnp
from jax import lax
from jax.experimental import pallas as pl
from jax.experimental.pallas import tpu as pltpu
from jax.experimental.pallas import tpu_sc as plsc

N_NODES = 10000
NPAD = 10240
D = 256
H = 128
E = 160000
NT = 16
CH = 128
NCH = 80
EPAD = NT * NCH * CH
RPT = NPAD // NT

_mesh = plsc.VectorSubcoreMesh(
    core_axis_name="c", subcore_axis_name="s", num_cores=2, num_subcores=NT
)


@functools.partial(
    pl.kernel,
    out_type=jax.ShapeDtypeStruct((2, NPAD), jnp.float32),
    mesh=_mesh,
    scratch_types=[
        pltpu.VMEM((NCH, CH), jnp.int32),
        pltpu.VMEM((CH,), jnp.float32),
        pltpu.VMEM((RPT,), jnp.float32),
        pltpu.VMEM_SHARED((NPAD,), jnp.float32),
    ],
)
def _sc_degrees(idx_hbm, out_hbm, idx_v, ones_v, zero_v, acc):
    c = lax.axis_index("c")
    s = lax.axis_index("s")

    def _setz(k, _):
        zero_v[pl.ds(k * 16, 16)] = jnp.zeros((16,), jnp.float32)
        return _

    lax.fori_loop(0, RPT // 16, _setz, None)

    def _seto(k, _):
        ones_v[pl.ds(k * 16, 16)] = jnp.ones((16,), jnp.float32)
        return _

    lax.fori_loop(0, CH // 16, _seto, None)

    pltpu.sync_copy(idx_hbm.at[c, s], idx_v)
    pltpu.sync_copy(zero_v, acc.at[pl.ds(s * RPT, RPT)])
    plsc.subcore_barrier()

    def _body(j, _):
        pltpu.sync_copy(ones_v, acc.at[idx_v.at[j]], add=True)
        return _

    lax.fori_loop(0, NCH, _body, None)
    plsc.subcore_barrier()
    pltpu.sync_copy(acc.at[pl.ds(s * RPT, RPT)], out_hbm.at[c, pl.ds(s * RPT, RPT)])


def _make_sc_aggregate(dtype):
  return functools.partial(
    pl.kernel,
    out_type=jax.ShapeDtypeStruct((2, NPAD, H), dtype),
    mesh=_mesh,
    scratch_types=[
        pltpu.VMEM((2, 4, 2, CH), jnp.int32),
        pltpu.VMEM((CH, H), dtype),
        pltpu.VMEM((CH, H), dtype),
        pltpu.SemaphoreType.DMA,
        pltpu.SemaphoreType.DMA,
        pltpu.SemaphoreType.DMA,
        pltpu.SemaphoreType.DMA,
        pltpu.SemaphoreType.DMA,
        pltpu.SemaphoreType.DMA,
        pltpu.VMEM_SHARED((NPAD, H), dtype),
    ],
  )(_sc_aggregate_body)


def _sc_aggregate_body(table_hbm, init_hbm, sridx_hbm, out_hbm,
                       idxr, rows0_v, rows1_v,
                       isem0, isem1, gsem0, gsem1,
                       ssem0, ssem1, acc):
    c = lax.axis_index("c")
    s = lax.axis_index("s")
    pltpu.sync_copy(init_hbm.at[c].at[pl.ds(s * RPT, RPT)],
                    acc.at[pl.ds(s * RPT, RPT)])

    table = table_hbm.at[c]
    isems = (isem0, isem1)
    rows = (rows0_v, rows1_v)
    gsems = (gsem0, gsem1)
    ssems = (ssem0, ssem1)

    def _idx_fetch(sl, j):
        pltpu.async_copy(sridx_hbm.at[s, pl.ds(j, 4)], idxr.at[sl],
                         isems[sl])

    def _idx_wait(sl):
        pltpu.make_async_copy(sridx_hbm.at[s, pl.ds(0, 4)], idxr.at[sl],
                              isems[sl]).wait()

    def _gather(sl, pos, k):
        pltpu.async_copy(table.at[idxr.at[sl, pos, 0]], rows[k], gsems[k])

    def _gather_wait(k):
        pltpu.make_async_copy(table.at[idxr.at[0, 0, 0]], rows[k],
                              gsems[k]).wait()

    def _scatter(sl, pos, k):
        pltpu.async_copy(rows[k], acc.at[idxr.at[sl, pos, 1]], ssems[k],
                         add=True)

    def _scatter_wait(k):
        pltpu.make_async_copy(rows[k], acc.at[idxr.at[0, 0, 1]],
                              ssems[k]).wait()

    _idx_fetch(0, 0)
    _idx_fetch(1, 4)
    plsc.subcore_barrier()
    _idx_wait(0)
    _gather(0, 0, 0)

    def _body(i, _):
        j = 8 * i
        for q in range(8):
            jq = j + q
            sl, pos, k = q // 4, q % 4, q % 2
            nk = (q + 1) % 2
            if q == 0:
                @pl.when(j > 0)
                def _():
                    _scatter_wait(nk)
                    _idx_fetch(1, j + 4)
            else:
                _scatter_wait(nk)
                if q == 4:
                    @pl.when(j + 8 < NCH)
                    def _():
                        _idx_fetch(0, j + 8)
            if q == 3:
                _idx_wait(1)
                _gather(1, 0, nk)
            elif q < 7:
                _gather(sl, pos + 1, nk)
            else:
                @pl.when(j + 8 < NCH)
                def _():
                    _idx_wait(0)
                    _gather(0, 0, nk)
            _gather_wait(k)
            _scatter(sl, pos, k)
        return _

    lax.fori_loop(0, NCH // 8, _body, None)
    _scatter_wait(1)
    plsc.subcore_barrier()
    pltpu.sync_copy(acc.at[pl.ds(s * RPT, RPT)],
                    out_hbm.at[c].at[pl.ds(s * RPT, RPT)])


_sc_aggregate_f32 = _make_sc_aggregate(jnp.float32)


_BR = 2048
_GRID = NPAD // _BR


def _tc1_body(x_ref, w_ref, b_ref, degs_ref, out_ref):
    h = jnp.dot(x_ref[...], w_ref[...], preferred_element_type=jnp.float32)
    h = h + b_ref[...]
    h = h * lax.rsqrt(jnp.maximum(degs_ref[...] + 1.0, 1.0))
    out_ref[0] = h[:, :H]
    out_ref[1] = h[:, H:]


def _tc2_body(a_ref, w_ref, b_ref, degr_ref, degs_ref, out_ref):
    t = jnp.concatenate([a_ref[0], a_ref[1]], axis=1)
    t = t * lax.rsqrt(jnp.maximum(degr_ref[...] + 1.0, 1.0))
    h = jnp.dot(t, w_ref[...], preferred_element_type=jnp.float32)
    h = h + b_ref[...]
    h = h * lax.rsqrt(jnp.maximum(degs_ref[...], 1.0))
    out_ref[0] = h[:, :H]
    out_ref[1] = h[:, H:]


def _tc3_body(a_ref, degr_ref, out_ref):
    t = jnp.concatenate([a_ref[0], a_ref[1]], axis=1)
    out_ref[...] = t * lax.rsqrt(jnp.maximum(degr_ref[...], 1.0))


_halves_spec = pl.BlockSpec((2, _BR, H), lambda i: (0, i, 0))
_rows_spec = pl.BlockSpec((_BR, D), lambda i: (i, 0))
_deg_spec = pl.BlockSpec((_BR, 1), lambda i: (i, 0))
_w_spec = pl.BlockSpec((D, D), lambda i: (0, 0))
_b_spec = pl.BlockSpec((1, D), lambda i: (0, 0))

_tc1 = pl.pallas_call(
    _tc1_body,
    grid=(_GRID,),
    in_specs=[_rows_spec, _w_spec, _b_spec, _deg_spec],
    out_specs=_halves_spec,
    out_shape=jax.ShapeDtypeStruct((2, NPAD, H), jnp.float32),
)

_tc2 = pl.pallas_call(
    _tc2_body,
    grid=(_GRID,),
    in_specs=[_halves_spec, _w_spec, _b_spec, _deg_spec, _deg_spec],
    out_specs=_halves_spec,
    out_shape=jax.ShapeDtypeStruct((2, NPAD, H), jnp.float32),
)

_tc3 = pl.pallas_call(
    _tc3_body,
    grid=(_GRID,),
    in_specs=[_halves_spec, _deg_spec],
    out_specs=_rows_spec,
    out_shape=jax.ShapeDtypeStruct((NPAD, D), jnp.float32),
)


def kernel(x, edge_index, W1, b1, W2, b2):
    ei = edge_index.astype(jnp.int32)
    senders, receivers = ei[0], ei[1]
    npad_e = EPAD - E
    dummy = jnp.full((npad_e,), N_NODES, dtype=jnp.int32)
    zpad = jnp.zeros((npad_e,), dtype=jnp.int32)

    hist_idx = jnp.stack([
        jnp.concatenate([senders, dummy]),
        jnp.concatenate([receivers, dummy]),
    ]).reshape(2, NT, NCH, CH)
    sidx = jnp.concatenate([senders, zpad]).reshape(NT, NCH, CH)
    ridx = jnp.concatenate([receivers, dummy]).reshape(NT, NCH, CH)
    sridx = jnp.stack([sidx, ridx], axis=2)

    x_pad = jnp.concatenate(
        [x, jnp.zeros((NPAD - N_NODES, D), dtype=jnp.float32)], axis=0
    )
    b1r = b1.reshape(1, D)
    b2r = b2.reshape(1, D)

    deg = _sc_degrees(hist_idx)
    deg_s = deg[0].reshape(NPAD, 1)
    deg_r = deg[1].reshape(NPAD, 1)

    hs = _tc1(x_pad, W1, b1r, deg_s)
    agg1 = _sc_aggregate_f32(hs, hs, sridx)
    h2s = _tc2(agg1, W2, b2r, deg_r, deg_s)
    zeros_init = jnp.zeros((2, NPAD, H), dtype=jnp.float32)
    agg2 = _sc_aggregate_f32(h2s, zeros_init, sridx)
    out = _tc3(agg2, deg_r)
    return out[:N_NODES]
